# baseline (speedup 1.0000x reference)
import functools

import jax
import jax.numpy as jnp
from jax import lax
from jax.experimental import pallas as pl
from jax.experimental.pallas import tpu as pltpu
from jax.experimental.pallas import tpu_sc as plsc

N = 10000
E = 320000
HID = 128
C = 40

NC = 2
NS = 16
NW = NC * NS

CHUNK = 128
CPW = 80
EPW = CPW * CHUNK
EP = NW * EPW
NP = N + 112
RPT = NP // NS
DW = 8

_sc_mesh = plsc.VectorSubcoreMesh(
    core_axis_name="c", subcore_axis_name="s", num_cores=NC, num_subcores=NS
)


@functools.partial(
    pl.kernel,
    out_type=jax.ShapeDtypeStruct((NC, NP, DW), jnp.float32),
    mesh=_sc_mesh,
    scratch_types=[
        pltpu.VMEM((CHUNK,), jnp.int32),
        pltpu.VMEM((CHUNK, DW), jnp.float32),
        pltpu.VMEM_SHARED((NP, DW), jnp.float32),
    ],
)
def _sc_degree(dst_hbm, ones_hbm, zeros_hbm, out_hbm, didx, ones_v, acc):
    c = lax.axis_index("c")
    s = lax.axis_index("s")
    w = c * NS + s
    pltpu.sync_copy(zeros_hbm, acc.at[pl.ds(s * RPT, RPT)])
    pltpu.sync_copy(ones_hbm, ones_v)
    plsc.subcore_barrier()
    base = w * CPW

    def body(j, carry):
        pltpu.sync_copy(dst_hbm.at[base + j], didx)
        pltpu.sync_copy(ones_v, acc.at[didx], add=True)
        return carry

    lax.fori_loop(0, CPW, body, 0)
    plsc.subcore_barrier()
    rows = pl.ds(s * RPT, RPT)
    pltpu.sync_copy(acc.at[rows], out_hbm.at[c, rows])


KB = 2
BPW = CPW // KB


@functools.partial(
    pl.kernel,
    out_type=jax.ShapeDtypeStruct((NC, NP, HID), jnp.float32),
    mesh=_sc_mesh,
    scratch_types=[
        pltpu.VMEM((KB, CHUNK), jnp.int32),
        pltpu.VMEM((KB, CHUNK), jnp.int32),
        pltpu.VMEM((KB * CHUNK, HID), jnp.float32),
        pltpu.VMEM_SHARED((NP, HID), jnp.float32),
        pltpu.SemaphoreType.DMA,
        pltpu.SemaphoreType.DMA,
    ],
)
def _sc_gather_scatter(zs_hbm, src_hbm, dst_hbm, zeros_hbm, out_hbm,
                       sidx, didx, rows_v, acc, sem_g, sem_s):
    c = lax.axis_index("c")
    s = lax.axis_index("s")
    w = c * NS + s
    pltpu.sync_copy(zeros_hbm, acc.at[pl.ds(s * RPT, RPT)])
    plsc.subcore_barrier()
    base = w * CPW

    def body(t, carry):
        row = base + t * KB
        pltpu.sync_copy(src_hbm.at[pl.ds(row, KB)], sidx)
        pltpu.sync_copy(dst_hbm.at[pl.ds(row, KB)], didx)
        for b in range(KB):
            pltpu.async_copy(
                zs_hbm.at[sidx.at[b]], rows_v.at[pl.ds(b * CHUNK, CHUNK)],
                sem_g,
            )
        for b in range(KB):
            pltpu.make_async_copy(
                zs_hbm.at[sidx.at[b]], rows_v.at[pl.ds(b * CHUNK, CHUNK)],
                sem_g,
            ).wait()
        for b in range(KB):
            pltpu.async_copy(
                rows_v.at[pl.ds(b * CHUNK, CHUNK)], acc.at[didx.at[b]],
                sem_s, add=True,
            )
        for b in range(KB):
            pltpu.make_async_copy(
                rows_v.at[pl.ds(b * CHUNK, CHUNK)], acc.at[didx.at[b]],
                sem_s,
            ).wait()
        return carry

    lax.fori_loop(0, BPW, body, 0)
    plsc.subcore_barrier()
    rows = pl.ds(s * RPT, RPT)
    pltpu.sync_copy(acc.at[rows], out_hbm.at[c, rows])


RB = 1000
GRID = N // RB


def _rows_spec(d=HID):
    return pl.BlockSpec((RB, d), lambda i: (i, 0))


def _full_spec(shape):
    return pl.BlockSpec(shape, lambda i: (0,) * len(shape))


def _deg_spec():
    return pl.BlockSpec((NC, RB, DW), lambda i: (0, i, 0))


def _layer_norm(x, g, b):
    mu = jnp.mean(x, axis=-1, keepdims=True)
    var = jnp.mean((x - mu) ** 2, axis=-1, keepdims=True)
    return (x - mu) * lax.rsqrt(var + 1e-5) * g + b


def _dinv_of(deg_ref):
    deg = deg_ref[0, :, 0] + deg_ref[1, :, 0] + 2.0
    return lax.rsqrt(deg)[:, None]


def _emb_body(x_ref, w_ref, b_ref, o_ref):
    o_ref[...] = (
        jnp.dot(x_ref[...], w_ref[...], preferred_element_type=jnp.float32)
        + b_ref[...]
    )


def _pre_body(h_ref, deg_ref, ng_ref, nb_ref, win_ref, bin_ref,
              sg_ref, sb_ref, wgcn_ref, u_ref, zs_ref):
    dinv = _dinv_of(deg_ref)
    t = _layer_norm(h_ref[...], ng_ref[...], nb_ref[...])
    a = (
        jnp.dot(t, win_ref[...], preferred_element_type=jnp.float32)
        + bin_ref[...]
    )
    u = 0.5 * a * (1.0 + lax.erf(a * 0.7071067811865476))
    g = _layer_norm(u, sg_ref[...], sb_ref[...])
    z = jnp.dot(g, wgcn_ref[...], preferred_element_type=jnp.float32)
    u_ref[...] = u
    zs_ref[...] = z * dinv


def _post_body(h_ref, u_ref, zs_ref, acc_ref, deg_ref, wout_ref, bout_ref,
               bgcn_ref, o_ref):
    dinv = _dinv_of(deg_ref)
    zs = zs_ref[...]
    gcn = dinv * (acc_ref[0] + acc_ref[1] + 2.0 * zs) + bgcn_ref[...]
    gated = jnp.tanh(gcn) * u_ref[...]
    o_ref[...] = (
        h_ref[...]
        + jnp.dot(gated, wout_ref[...], preferred_element_type=jnp.float32)
        + bout_ref[...]
    )


def _final_body(h_ref, w_ref, b_ref, o_ref):
    o_ref[...] = (
        jnp.dot(h_ref[...], w_ref[...], preferred_element_type=jnp.float32)
        + b_ref[...]
    )


_emb = pl.pallas_call(
    _emb_body,
    grid=(GRID,),
    in_specs=[_rows_spec(), _full_spec((HID, HID)), _full_spec((1, HID))],
    out_specs=_rows_spec(),
    out_shape=jax.ShapeDtypeStruct((N, HID), jnp.float32),
)

_pre = pl.pallas_call(
    _pre_body,
    grid=(GRID,),
    in_specs=[
        _rows_spec(), _deg_spec(),
        _full_spec((1, HID)), _full_spec((1, HID)),
        _full_spec((HID, HID)), _full_spec((1, HID)),
        _full_spec((1, HID)), _full_spec((1, HID)),
        _full_spec((HID, HID)),
    ],
    out_specs=[_rows_spec(), _rows_spec()],
    out_shape=[
        jax.ShapeDtypeStruct((N, HID), jnp.float32),
        jax.ShapeDtypeStruct((N, HID), jnp.float32),
    ],
)

_post = pl.pallas_call(
    _post_body,
    grid=(GRID,),
    in_specs=[
        _rows_spec(), _rows_spec(), _rows_spec(),
        pl.BlockSpec((NC, RB, HID), lambda i: (0, i, 0)),
        _deg_spec(),
        _full_spec((HID, HID)), _full_spec((1, HID)), _full_spec((1, HID)),
    ],
    out_specs=_rows_spec(),
    out_shape=jax.ShapeDtypeStruct((N, HID), jnp.float32),
)

_final = pl.pallas_call(
    _final_body,
    grid=(GRID,),
    in_specs=[_rows_spec(), _full_spec((HID, C)), _full_spec((1, C))],
    out_specs=_rows_spec(C),
    out_shape=jax.ShapeDtypeStruct((N, C), jnp.float32),
)


def kernel(x, params, edge_index):
    f32 = jnp.float32
    src = edge_index[0].astype(jnp.int32)
    dst = edge_index[1].astype(jnp.int32)
    pad = EP - E
    srcp = jnp.concatenate([src, jnp.zeros((pad,), jnp.int32)]).reshape(
        EP // CHUNK, CHUNK)
    dstp = jnp.concatenate([dst, jnp.full((pad,), N, jnp.int32)]).reshape(
        EP // CHUNK, CHUNK)

    ones_dw = jnp.ones((CHUNK, DW), f32)
    zeros_dw = jnp.zeros((RPT, DW), f32)
    zeros_h = jnp.zeros((RPT, HID), f32)

    degp = _sc_degree(dstp, ones_dw, zeros_dw)[:, :N, :]

    p = params
    h = _emb(x, p['Wemb'].T, p['bemb'][None, :])
    for lp in p['layers']:
        u, zs = _pre(
            h, degp,
            lp['norm_g'][None, :], lp['norm_b'][None, :],
            lp['Win'].T, lp['bin'][None, :],
            lp['sgu_norm_g'][None, :], lp['sgu_norm_b'][None, :],
            lp['Wgcn'].T,
        )
        acc = _sc_gather_scatter(zs, srcp, dstp, zeros_h)[:, :N, :]
        h = _post(
            h, u, zs, acc, degp,
            lp['Wout'].T, lp['bout'][None, :], lp['bgcn'][None, :],
        )
    return _final(h, p['Wlin'].T, p['blin'][None, :])


# speedup vs baseline: 1.0198x; 1.0198x over previous
import functools

import jax
import jax.numpy as jnp
from jax import lax
from jax.experimental import pallas as pl
from jax.experimental.pallas import tpu as pltpu
from jax.experimental.pallas import tpu_sc as plsc

N = 10000
E = 320000
HID = 128
C = 40

NC = 2
NS = 16
NW = NC * NS

CHUNK = 128
CPW = 80
EPW = CPW * CHUNK
EP = NW * EPW
NP = N + 112
RPT = NP // NS
DW = 8

_sc_mesh = plsc.VectorSubcoreMesh(
    core_axis_name="c", subcore_axis_name="s", num_cores=NC, num_subcores=NS
)


@functools.partial(
    pl.kernel,
    out_type=jax.ShapeDtypeStruct((NC, NP, DW), jnp.float32),
    mesh=_sc_mesh,
    scratch_types=[
        pltpu.VMEM((CHUNK,), jnp.int32),
        pltpu.VMEM((CHUNK, DW), jnp.float32),
        pltpu.VMEM_SHARED((NP, DW), jnp.float32),
    ],
)
def _sc_degree(dst_hbm, ones_hbm, zeros_hbm, out_hbm, didx, ones_v, acc):
    c = lax.axis_index("c")
    s = lax.axis_index("s")
    w = c * NS + s
    pltpu.sync_copy(zeros_hbm, acc.at[pl.ds(s * RPT, RPT)])
    pltpu.sync_copy(ones_hbm, ones_v)
    plsc.subcore_barrier()
    base = w * CPW

    def body(j, carry):
        pltpu.sync_copy(dst_hbm.at[base + j], didx)
        pltpu.sync_copy(ones_v, acc.at[didx], add=True)
        return carry

    lax.fori_loop(0, CPW, body, 0)
    plsc.subcore_barrier()
    rows = pl.ds(s * RPT, RPT)
    pltpu.sync_copy(acc.at[rows], out_hbm.at[c, rows])


@functools.partial(
    pl.kernel,
    out_type=jax.ShapeDtypeStruct((NC, NP, HID), jnp.float32),
    mesh=_sc_mesh,
    scratch_types=[
        pltpu.VMEM((CPW, CHUNK), jnp.int32),
        pltpu.VMEM((CPW, CHUNK), jnp.int32),
        pltpu.VMEM((CHUNK, HID), jnp.float32),
        pltpu.VMEM_SHARED((NP, HID), jnp.float32),
        pltpu.SemaphoreType.DMA,
    ],
)
def _sc_gather_scatter(zs_hbm, src_hbm, dst_hbm, zeros_hbm, out_hbm,
                       sidx, didx, rows_v, acc, sem_g):
    c = lax.axis_index("c")
    s = lax.axis_index("s")
    w = c * NS + s
    pltpu.sync_copy(zeros_hbm, acc.at[pl.ds(s * RPT, RPT)])
    base = w * CPW
    pltpu.sync_copy(src_hbm.at[pl.ds(base, CPW)], sidx)
    pltpu.sync_copy(dst_hbm.at[pl.ds(base, CPW)], didx)
    plsc.subcore_barrier()

    def body(j, carry):
        pltpu.async_copy(zs_hbm.at[sidx.at[j]], rows_v, sem_g).wait()
        pltpu.sync_copy(rows_v, acc.at[didx.at[j]], add=True)
        return carry

    lax.fori_loop(0, CPW, body, 0)
    plsc.subcore_barrier()
    rows = pl.ds(s * RPT, RPT)
    pltpu.sync_copy(acc.at[rows], out_hbm.at[c, rows])


RB = 1000
GRID = N // RB


def _rows_spec(d=HID):
    return pl.BlockSpec((RB, d), lambda i: (i, 0))


def _full_spec(shape):
    return pl.BlockSpec(shape, lambda i: (0,) * len(shape))


def _deg_spec():
    return pl.BlockSpec((NC, RB, DW), lambda i: (0, i, 0))


def _layer_norm(x, g, b):
    mu = jnp.mean(x, axis=-1, keepdims=True)
    var = jnp.mean((x - mu) ** 2, axis=-1, keepdims=True)
    return (x - mu) * lax.rsqrt(var + 1e-5) * g + b


def _dinv_of(deg_ref):
    deg = deg_ref[0, :, 0] + deg_ref[1, :, 0] + 2.0
    return lax.rsqrt(deg)[:, None]


def _emb_body(x_ref, w_ref, b_ref, o_ref):
    o_ref[...] = (
        jnp.dot(x_ref[...], w_ref[...], preferred_element_type=jnp.float32)
        + b_ref[...]
    )


def _pre_body(h_ref, deg_ref, ng_ref, nb_ref, win_ref, bin_ref,
              sg_ref, sb_ref, wgcn_ref, u_ref, zs_ref):
    dinv = _dinv_of(deg_ref)
    t = _layer_norm(h_ref[...], ng_ref[...], nb_ref[...])
    a = (
        jnp.dot(t, win_ref[...], preferred_element_type=jnp.float32)
        + bin_ref[...]
    )
    u = 0.5 * a * (1.0 + lax.erf(a * 0.7071067811865476))
    g = _layer_norm(u, sg_ref[...], sb_ref[...])
    z = jnp.dot(g, wgcn_ref[...], preferred_element_type=jnp.float32)
    u_ref[...] = u
    zs_ref[...] = z * dinv


def _post_body(h_ref, u_ref, zs_ref, acc_ref, deg_ref, wout_ref, bout_ref,
               bgcn_ref, o_ref):
    dinv = _dinv_of(deg_ref)
    zs = zs_ref[...]
    gcn = dinv * (acc_ref[0] + acc_ref[1] + 2.0 * zs) + bgcn_ref[...]
    gated = jnp.tanh(gcn) * u_ref[...]
    o_ref[...] = (
        h_ref[...]
        + jnp.dot(gated, wout_ref[...], preferred_element_type=jnp.float32)
        + bout_ref[...]
    )


def _final_body(h_ref, w_ref, b_ref, o_ref):
    o_ref[...] = (
        jnp.dot(h_ref[...], w_ref[...], preferred_element_type=jnp.float32)
        + b_ref[...]
    )


_emb = pl.pallas_call(
    _emb_body,
    grid=(GRID,),
    in_specs=[_rows_spec(), _full_spec((HID, HID)), _full_spec((1, HID))],
    out_specs=_rows_spec(),
    out_shape=jax.ShapeDtypeStruct((N, HID), jnp.float32),
)

_pre = pl.pallas_call(
    _pre_body,
    grid=(GRID,),
    in_specs=[
        _rows_spec(), _deg_spec(),
        _full_spec((1, HID)), _full_spec((1, HID)),
        _full_spec((HID, HID)), _full_spec((1, HID)),
        _full_spec((1, HID)), _full_spec((1, HID)),
        _full_spec((HID, HID)),
    ],
    out_specs=[_rows_spec(), _rows_spec()],
    out_shape=[
        jax.ShapeDtypeStruct((N, HID), jnp.float32),
        jax.ShapeDtypeStruct((N, HID), jnp.float32),
    ],
)

_post = pl.pallas_call(
    _post_body,
    grid=(GRID,),
    in_specs=[
        _rows_spec(), _rows_spec(), _rows_spec(),
        pl.BlockSpec((NC, RB, HID), lambda i: (0, i, 0)),
        _deg_spec(),
        _full_spec((HID, HID)), _full_spec((1, HID)), _full_spec((1, HID)),
    ],
    out_specs=_rows_spec(),
    out_shape=jax.ShapeDtypeStruct((N, HID), jnp.float32),
)

_final = pl.pallas_call(
    _final_body,
    grid=(GRID,),
    in_specs=[_rows_spec(), _full_spec((HID, C)), _full_spec((1, C))],
    out_specs=_rows_spec(C),
    out_shape=jax.ShapeDtypeStruct((N, C), jnp.float32),
)


def kernel(x, params, edge_index):
    f32 = jnp.float32
    src = edge_index[0].astype(jnp.int32)
    dst = edge_index[1].astype(jnp.int32)
    pad = EP - E
    srcp = jnp.concatenate([src, jnp.zeros((pad,), jnp.int32)]).reshape(
        EP // CHUNK, CHUNK)
    dstp = jnp.concatenate([dst, jnp.full((pad,), N, jnp.int32)]).reshape(
        EP // CHUNK, CHUNK)

    ones_dw = jnp.ones((CHUNK, DW), f32)
    zeros_dw = jnp.zeros((RPT, DW), f32)
    zeros_h = jnp.zeros((RPT, HID), f32)

    degp = _sc_degree(dstp, ones_dw, zeros_dw)[:, :N, :]

    p = params
    h = _emb(x, p['Wemb'].T, p['bemb'][None, :])
    for lp in p['layers']:
        u, zs = _pre(
            h, degp,
            lp['norm_g'][None, :], lp['norm_b'][None, :],
            lp['Win'].T, lp['bin'][None, :],
            lp['sgu_norm_g'][None, :], lp['sgu_norm_b'][None, :],
            lp['Wgcn'].T,
        )
        acc = _sc_gather_scatter(zs, srcp, dstp, zeros_h)[:, :N, :]
        h = _post(
            h, u, zs, acc, degp,
            lp['Wout'].T, lp['bout'][None, :], lp['bgcn'][None, :],
        )
    return _final(h, p['Wlin'].T, p['blin'][None, :])


# speedup vs baseline: 1.2757x; 1.2510x over previous
import functools

import jax
import jax.numpy as jnp
from jax import lax
from jax.experimental import pallas as pl
from jax.experimental.pallas import tpu as pltpu
from jax.experimental.pallas import tpu_sc as plsc

N = 10000
E = 320000
HID = 128
C = 40

NC = 2
NS = 16
NW = NC * NS

CHUNK = 128
CPW0 = 62
CPW1 = 96
NCH = NS * (CPW0 + CPW1)
EP = NCH * CHUNK
NP = N + 112
RPT = NP // NS
DW = 8

_sc_mesh = plsc.VectorSubcoreMesh(
    core_axis_name="c", subcore_axis_name="s", num_cores=NC, num_subcores=NS
)


@functools.partial(
    pl.kernel,
    out_type=jax.ShapeDtypeStruct((NC, NP, DW), jnp.float32),
    mesh=_sc_mesh,
    scratch_types=[
        pltpu.VMEM((CHUNK,), jnp.int32),
        pltpu.VMEM((CHUNK, DW), jnp.float32),
        pltpu.VMEM_SHARED((NP, DW), jnp.float32),
    ],
)
def _sc_degree(dst_hbm, ones_hbm, zeros_hbm, out_hbm, didx, ones_v, acc):
    c = lax.axis_index("c")
    s = lax.axis_index("s")
    pltpu.sync_copy(zeros_hbm, acc.at[pl.ds(s * RPT, RPT)])
    pltpu.sync_copy(ones_hbm, ones_v)
    plsc.subcore_barrier()
    base = jnp.where(c == 0, s * CPW0, NS * CPW0 + s * CPW1)
    nb = jnp.where(c == 0, CPW0, CPW1)

    def body(j, carry):
        pltpu.sync_copy(dst_hbm.at[base + j], didx)
        pltpu.sync_copy(ones_v, acc.at[didx], add=True)
        return carry

    lax.fori_loop(0, nb, body, 0)
    plsc.subcore_barrier()
    rows = pl.ds(s * RPT, RPT)
    pltpu.sync_copy(acc.at[rows], out_hbm.at[c, rows])


@functools.partial(
    pl.kernel,
    out_type=jax.ShapeDtypeStruct((NC, NP, HID), jnp.float32),
    mesh=_sc_mesh,
    scratch_types=[
        pltpu.VMEM((CHUNK,), jnp.int32),
        pltpu.VMEM((CHUNK,), jnp.int32),
        pltpu.VMEM((CHUNK, HID), jnp.float32),
        pltpu.VMEM_SHARED((NP, HID), jnp.float32),
        pltpu.SemaphoreType.DMA,
    ],
)
def _sc_gather_scatter(zs_hbm, src_hbm, dst_hbm, zeros_hbm, out_hbm,
                       sidx, didx, rows_v, acc, sem_g):
    c = lax.axis_index("c")
    s = lax.axis_index("s")
    pltpu.sync_copy(zeros_hbm, acc.at[pl.ds(s * RPT, RPT)])
    plsc.subcore_barrier()
    base = jnp.where(c == 0, s * CPW0, NS * CPW0 + s * CPW1)
    nb = jnp.where(c == 0, CPW0, CPW1)

    def body(j, carry):
        pltpu.sync_copy(src_hbm.at[base + j], sidx)
        pltpu.sync_copy(dst_hbm.at[base + j], didx)
        pltpu.async_copy(zs_hbm.at[sidx], rows_v, sem_g).wait()
        pltpu.sync_copy(rows_v, acc.at[didx], add=True)
        return carry

    lax.fori_loop(0, nb, body, 0)
    plsc.subcore_barrier()
    rows = pl.ds(s * RPT, RPT)
    pltpu.sync_copy(acc.at[rows], out_hbm.at[c, rows])


RB = 1000
GRID = N // RB


def _rows_spec(d=HID):
    return pl.BlockSpec((RB, d), lambda i: (i, 0))


def _full_spec(shape):
    return pl.BlockSpec(shape, lambda i: (0,) * len(shape))


def _deg_spec():
    return pl.BlockSpec((NC, RB, DW), lambda i: (0, i, 0))


def _layer_norm(x, g, b):
    mu = jnp.mean(x, axis=-1, keepdims=True)
    var = jnp.mean((x - mu) ** 2, axis=-1, keepdims=True)
    return (x - mu) * lax.rsqrt(var + 1e-5) * g + b


def _dinv_of(deg_ref):
    deg = deg_ref[0, :, 0] + deg_ref[1, :, 0] + 2.0
    return lax.rsqrt(deg)[:, None]


def _emb_body(x_ref, w_ref, b_ref, o_ref):
    o_ref[...] = (
        jnp.dot(x_ref[...], w_ref[...], preferred_element_type=jnp.float32)
        + b_ref[...]
    )


def _pre_body(h_ref, deg_ref, ng_ref, nb_ref, win_ref, bin_ref,
              sg_ref, sb_ref, wgcn_ref, u_ref, zs_ref):
    dinv = _dinv_of(deg_ref)
    t = _layer_norm(h_ref[...], ng_ref[...], nb_ref[...])
    a = (
        jnp.dot(t, win_ref[...], preferred_element_type=jnp.float32)
        + bin_ref[...]
    )
    u = 0.5 * a * (1.0 + lax.erf(a * 0.7071067811865476))
    g = _layer_norm(u, sg_ref[...], sb_ref[...])
    z = jnp.dot(g, wgcn_ref[...], preferred_element_type=jnp.float32)
    u_ref[...] = u
    zs_ref[...] = z * dinv


def _post_body(h_ref, u_ref, zs_ref, acc_ref, deg_ref, wout_ref, bout_ref,
               bgcn_ref, o_ref):
    dinv = _dinv_of(deg_ref)
    zs = zs_ref[...]
    gcn = dinv * (acc_ref[0] + acc_ref[1] + 2.0 * zs) + bgcn_ref[...]
    gated = jnp.tanh(gcn) * u_ref[...]
    o_ref[...] = (
        h_ref[...]
        + jnp.dot(gated, wout_ref[...], preferred_element_type=jnp.float32)
        + bout_ref[...]
    )


def _final_body(h_ref, w_ref, b_ref, o_ref):
    o_ref[...] = (
        jnp.dot(h_ref[...], w_ref[...], preferred_element_type=jnp.float32)
        + b_ref[...]
    )


_emb = pl.pallas_call(
    _emb_body,
    grid=(GRID,),
    in_specs=[_rows_spec(), _full_spec((HID, HID)), _full_spec((1, HID))],
    out_specs=_rows_spec(),
    out_shape=jax.ShapeDtypeStruct((N, HID), jnp.float32),
)

_pre = pl.pallas_call(
    _pre_body,
    grid=(GRID,),
    in_specs=[
        _rows_spec(), _deg_spec(),
        _full_spec((1, HID)), _full_spec((1, HID)),
        _full_spec((HID, HID)), _full_spec((1, HID)),
        _full_spec((1, HID)), _full_spec((1, HID)),
        _full_spec((HID, HID)),
    ],
    out_specs=[_rows_spec(), _rows_spec()],
    out_shape=[
        jax.ShapeDtypeStruct((N, HID), jnp.float32),
        jax.ShapeDtypeStruct((N, HID), jnp.float32),
    ],
)

_post = pl.pallas_call(
    _post_body,
    grid=(GRID,),
    in_specs=[
        _rows_spec(), _rows_spec(), _rows_spec(),
        pl.BlockSpec((NC, RB, HID), lambda i: (0, i, 0)),
        _deg_spec(),
        _full_spec((HID, HID)), _full_spec((1, HID)), _full_spec((1, HID)),
    ],
    out_specs=_rows_spec(),
    out_shape=jax.ShapeDtypeStruct((N, HID), jnp.float32),
)

_final = pl.pallas_call(
    _final_body,
    grid=(GRID,),
    in_specs=[_rows_spec(), _full_spec((HID, C)), _full_spec((1, C))],
    out_specs=_rows_spec(C),
    out_shape=jax.ShapeDtypeStruct((N, C), jnp.float32),
)


def kernel(x, params, edge_index):
    f32 = jnp.float32
    src = edge_index[0].astype(jnp.int32)
    dst = edge_index[1].astype(jnp.int32)
    pad = EP - E
    srcp = jnp.concatenate([src, jnp.zeros((pad,), jnp.int32)]).reshape(
        EP // CHUNK, CHUNK)
    dstp = jnp.concatenate([dst, jnp.full((pad,), N, jnp.int32)]).reshape(
        EP // CHUNK, CHUNK)

    ones_dw = jnp.ones((CHUNK, DW), f32)
    zeros_dw = jnp.zeros((RPT, DW), f32)
    zeros_h = jnp.zeros((RPT, HID), f32)

    degp = _sc_degree(dstp, ones_dw, zeros_dw)[:, :N, :]

    p = params
    h = _emb(x, p['Wemb'].T, p['bemb'][None, :])
    for lp in p['layers']:
        u, zs = _pre(
            h, degp,
            lp['norm_g'][None, :], lp['norm_b'][None, :],
            lp['Win'].T, lp['bin'][None, :],
            lp['sgu_norm_g'][None, :], lp['sgu_norm_b'][None, :],
            lp['Wgcn'].T,
        )
        acc = _sc_gather_scatter(zs, srcp, dstp, zeros_h)[:, :N, :]
        h = _post(
            h, u, zs, acc, degp,
            lp['Wout'].T, lp['bout'][None, :], lp['bgcn'][None, :],
        )
    return _final(h, p['Wlin'].T, p['blin'][None, :])


# speedup vs baseline: 1.3974x; 1.0954x over previous
import functools

import jax
import jax.numpy as jnp
from jax import lax
from jax.experimental import pallas as pl
from jax.experimental.pallas import tpu as pltpu
from jax.experimental.pallas import tpu_sc as plsc

N = 10000
E = 320000
HID = 128
C = 40

NC = 2
NS = 16
NW = NC * NS

CHUNK = 128
CPW0 = 79
CPW1 = 79
NCH = NS * (CPW0 + CPW1)
EP = NCH * CHUNK
NP = N + 112
RPT = NP // NS
DW = 8

_sc_mesh = plsc.VectorSubcoreMesh(
    core_axis_name="c", subcore_axis_name="s", num_cores=NC, num_subcores=NS
)


@functools.partial(
    pl.kernel,
    out_type=jax.ShapeDtypeStruct((NC, NP, DW), jnp.float32),
    mesh=_sc_mesh,
    scratch_types=[
        pltpu.VMEM((CHUNK,), jnp.int32),
        pltpu.VMEM((CHUNK, DW), jnp.float32),
        pltpu.VMEM_SHARED((NP, DW), jnp.float32),
    ],
)
def _sc_degree(dst_hbm, ones_hbm, zeros_hbm, out_hbm, didx, ones_v, acc):
    c = lax.axis_index("c")
    s = lax.axis_index("s")
    pltpu.sync_copy(zeros_hbm, acc.at[pl.ds(s * RPT, RPT)])
    pltpu.sync_copy(ones_hbm, ones_v)
    plsc.subcore_barrier()
    base = jnp.where(c == 0, s * CPW0, NS * CPW0 + s * CPW1)
    nb = jnp.where(c == 0, CPW0, CPW1)

    def body(j, carry):
        pltpu.sync_copy(dst_hbm.at[base + j], didx)
        pltpu.sync_copy(ones_v, acc.at[didx], add=True)
        return carry

    lax.fori_loop(0, nb, body, 0)
    plsc.subcore_barrier()
    rows = pl.ds(s * RPT, RPT)
    pltpu.sync_copy(acc.at[rows], out_hbm.at[c, rows])


@functools.partial(
    pl.kernel,
    out_type=jax.ShapeDtypeStruct((NC, NP, HID), jnp.float32),
    mesh=_sc_mesh,
    scratch_types=[
        pltpu.VMEM((CHUNK,), jnp.int32),
        pltpu.VMEM((CHUNK,), jnp.int32),
        pltpu.VMEM((CHUNK, HID), jnp.float32),
        pltpu.VMEM_SHARED((NP, HID), jnp.float32),
        pltpu.SemaphoreType.DMA,
    ],
)
def _sc_gather_scatter(zs_hbm, src_hbm, dst_hbm, zeros_hbm, out_hbm,
                       sidx, didx, rows_v, acc, sem_g):
    c = lax.axis_index("c")
    s = lax.axis_index("s")
    pltpu.sync_copy(zeros_hbm, acc.at[pl.ds(s * RPT, RPT)])
    plsc.subcore_barrier()
    base = jnp.where(c == 0, s * CPW0, NS * CPW0 + s * CPW1)
    nb = jnp.where(c == 0, CPW0, CPW1)

    def body(j, carry):
        pltpu.sync_copy(src_hbm.at[base + j], sidx)
        pltpu.sync_copy(dst_hbm.at[base + j], didx)
        pltpu.async_copy(zs_hbm.at[sidx], rows_v, sem_g).wait()
        pltpu.sync_copy(rows_v, acc.at[didx], add=True)
        return carry

    lax.fori_loop(0, nb, body, 0)
    plsc.subcore_barrier()
    rows = pl.ds(s * RPT, RPT)
    pltpu.sync_copy(acc.at[rows], out_hbm.at[c, rows])


RB = 1000
GRID = N // RB


def _rows_spec(d=HID):
    return pl.BlockSpec((RB, d), lambda i: (i, 0))


def _full_spec(shape):
    return pl.BlockSpec(shape, lambda i: (0,) * len(shape))


def _deg_spec():
    return pl.BlockSpec((NC, RB, DW), lambda i: (0, i, 0))


def _layer_norm(x, g, b):
    mu = jnp.mean(x, axis=-1, keepdims=True)
    var = jnp.mean((x - mu) ** 2, axis=-1, keepdims=True)
    return (x - mu) * lax.rsqrt(var + 1e-5) * g + b


def _dinv_of(deg_ref):
    deg = deg_ref[0, :, 0] + deg_ref[1, :, 0] + 2.0
    return lax.rsqrt(deg)[:, None]


def _emb_body(x_ref, w_ref, b_ref, o_ref):
    o_ref[...] = (
        jnp.dot(x_ref[...], w_ref[...], preferred_element_type=jnp.float32)
        + b_ref[...]
    )


def _pre_body(h_ref, deg_ref, ng_ref, nb_ref, win_ref, bin_ref,
              sg_ref, sb_ref, wgcn_ref, u_ref, zs_ref):
    dinv = _dinv_of(deg_ref)
    t = _layer_norm(h_ref[...], ng_ref[...], nb_ref[...])
    a = (
        jnp.dot(t, win_ref[...], preferred_element_type=jnp.float32)
        + bin_ref[...]
    )
    u = 0.5 * a * (1.0 + lax.erf(a * 0.7071067811865476))
    g = _layer_norm(u, sg_ref[...], sb_ref[...])
    z = jnp.dot(g, wgcn_ref[...], preferred_element_type=jnp.float32)
    u_ref[...] = u
    zs_ref[...] = z * dinv


def _post_body(h_ref, u_ref, zs_ref, acc_ref, deg_ref, wout_ref, bout_ref,
               bgcn_ref, o_ref):
    dinv = _dinv_of(deg_ref)
    zs = zs_ref[...]
    gcn = dinv * (acc_ref[0] + acc_ref[1] + 2.0 * zs) + bgcn_ref[...]
    gated = jnp.tanh(gcn) * u_ref[...]
    o_ref[...] = (
        h_ref[...]
        + jnp.dot(gated, wout_ref[...], preferred_element_type=jnp.float32)
        + bout_ref[...]
    )


def _final_body(h_ref, w_ref, b_ref, o_ref):
    o_ref[...] = (
        jnp.dot(h_ref[...], w_ref[...], preferred_element_type=jnp.float32)
        + b_ref[...]
    )


_emb = pl.pallas_call(
    _emb_body,
    grid=(GRID,),
    in_specs=[_rows_spec(), _full_spec((HID, HID)), _full_spec((1, HID))],
    out_specs=_rows_spec(),
    out_shape=jax.ShapeDtypeStruct((N, HID), jnp.float32),
)

_pre = pl.pallas_call(
    _pre_body,
    grid=(GRID,),
    in_specs=[
        _rows_spec(), _deg_spec(),
        _full_spec((1, HID)), _full_spec((1, HID)),
        _full_spec((HID, HID)), _full_spec((1, HID)),
        _full_spec((1, HID)), _full_spec((1, HID)),
        _full_spec((HID, HID)),
    ],
    out_specs=[_rows_spec(), _rows_spec()],
    out_shape=[
        jax.ShapeDtypeStruct((N, HID), jnp.float32),
        jax.ShapeDtypeStruct((N, HID), jnp.float32),
    ],
)

_post = pl.pallas_call(
    _post_body,
    grid=(GRID,),
    in_specs=[
        _rows_spec(), _rows_spec(), _rows_spec(),
        pl.BlockSpec((NC, RB, HID), lambda i: (0, i, 0)),
        _deg_spec(),
        _full_spec((HID, HID)), _full_spec((1, HID)), _full_spec((1, HID)),
    ],
    out_specs=_rows_spec(),
    out_shape=jax.ShapeDtypeStruct((N, HID), jnp.float32),
)

_final = pl.pallas_call(
    _final_body,
    grid=(GRID,),
    in_specs=[_rows_spec(), _full_spec((HID, C)), _full_spec((1, C))],
    out_specs=_rows_spec(C),
    out_shape=jax.ShapeDtypeStruct((N, C), jnp.float32),
)


def kernel(x, params, edge_index):
    f32 = jnp.float32
    src = edge_index[0].astype(jnp.int32)
    dst = edge_index[1].astype(jnp.int32)
    pad = EP - E
    srcp = jnp.concatenate([src, jnp.zeros((pad,), jnp.int32)]).reshape(
        EP // CHUNK, CHUNK)
    dstp = jnp.concatenate([dst, jnp.full((pad,), N, jnp.int32)]).reshape(
        EP // CHUNK, CHUNK)

    ones_dw = jnp.ones((CHUNK, DW), f32)
    zeros_dw = jnp.zeros((RPT, DW), f32)
    zeros_h = jnp.zeros((RPT, HID), f32)

    degp = _sc_degree(dstp, ones_dw, zeros_dw)[:, :N, :]

    p = params
    h = _emb(x, p['Wemb'].T, p['bemb'][None, :])
    for lp in p['layers']:
        u, zs = _pre(
            h, degp,
            lp['norm_g'][None, :], lp['norm_b'][None, :],
            lp['Win'].T, lp['bin'][None, :],
            lp['sgu_norm_g'][None, :], lp['sgu_norm_b'][None, :],
            lp['Wgcn'].T,
        )
        acc = _sc_gather_scatter(zs, srcp, dstp, zeros_h)[:, :N, :]
        h = _post(
            h, u, zs, acc, degp,
            lp['Wout'].T, lp['bout'][None, :], lp['bgcn'][None, :],
        )
    return _final(h, p['Wlin'].T, p['blin'][None, :])


# speedup vs baseline: 1.5228x; 1.0897x over previous
import functools

import jax
import jax.numpy as jnp
from jax import lax
from jax.experimental import pallas as pl
from jax.experimental.pallas import tpu as pltpu
from jax.experimental.pallas import tpu_sc as plsc

N = 10000
E = 320000
HID = 128
C = 40

NC = 2
NS = 16
NW = NC * NS

CHUNK = 128
CPW0 = 96
CPW1 = 62
NCH = NS * (CPW0 + CPW1)
EP = NCH * CHUNK
NP = N + 112
RPT = NP // NS
DW = 8

_sc_mesh = plsc.VectorSubcoreMesh(
    core_axis_name="c", subcore_axis_name="s", num_cores=NC, num_subcores=NS
)


@functools.partial(
    pl.kernel,
    out_type=jax.ShapeDtypeStruct((NC, NP, DW), jnp.float32),
    mesh=_sc_mesh,
    scratch_types=[
        pltpu.VMEM((CHUNK,), jnp.int32),
        pltpu.VMEM((CHUNK, DW), jnp.float32),
        pltpu.VMEM_SHARED((NP, DW), jnp.float32),
    ],
)
def _sc_degree(dst_hbm, ones_hbm, zeros_hbm, out_hbm, didx, ones_v, acc):
    c = lax.axis_index("c")
    s = lax.axis_index("s")
    pltpu.sync_copy(zeros_hbm, acc.at[pl.ds(s * RPT, RPT)])
    pltpu.sync_copy(ones_hbm, ones_v)
    plsc.subcore_barrier()
    base = jnp.where(c == 0, s * CPW0, NS * CPW0 + s * CPW1)
    nb = jnp.where(c == 0, CPW0, CPW1)

    def body(j, carry):
        pltpu.sync_copy(dst_hbm.at[base + j], didx)
        pltpu.sync_copy(ones_v, acc.at[didx], add=True)
        return carry

    lax.fori_loop(0, nb, body, 0)
    plsc.subcore_barrier()
    rows = pl.ds(s * RPT, RPT)
    pltpu.sync_copy(acc.at[rows], out_hbm.at[c, rows])


@functools.partial(
    pl.kernel,
    out_type=jax.ShapeDtypeStruct((NC, NP, HID), jnp.float32),
    mesh=_sc_mesh,
    scratch_types=[
        pltpu.VMEM((CHUNK,), jnp.int32),
        pltpu.VMEM((CHUNK,), jnp.int32),
        pltpu.VMEM((CHUNK, HID), jnp.float32),
        pltpu.VMEM_SHARED((NP, HID), jnp.float32),
        pltpu.SemaphoreType.DMA,
    ],
)
def _sc_gather_scatter(zs_hbm, src_hbm, dst_hbm, zeros_hbm, out_hbm,
                       sidx, didx, rows_v, acc, sem_g):
    c = lax.axis_index("c")
    s = lax.axis_index("s")
    pltpu.sync_copy(zeros_hbm, acc.at[pl.ds(s * RPT, RPT)])
    plsc.subcore_barrier()
    base = jnp.where(c == 0, s * CPW0, NS * CPW0 + s * CPW1)
    nb = jnp.where(c == 0, CPW0, CPW1)

    def body(j, carry):
        pltpu.sync_copy(src_hbm.at[base + j], sidx)
        pltpu.sync_copy(dst_hbm.at[base + j], didx)
        pltpu.async_copy(zs_hbm.at[sidx], rows_v, sem_g).wait()
        pltpu.sync_copy(rows_v, acc.at[didx], add=True)
        return carry

    lax.fori_loop(0, nb, body, 0)
    plsc.subcore_barrier()
    rows = pl.ds(s * RPT, RPT)
    pltpu.sync_copy(acc.at[rows], out_hbm.at[c, rows])


RB = 1000
GRID = N // RB


def _rows_spec(d=HID):
    return pl.BlockSpec((RB, d), lambda i: (i, 0))


def _full_spec(shape):
    return pl.BlockSpec(shape, lambda i: (0,) * len(shape))


def _deg_spec():
    return pl.BlockSpec((NC, RB, DW), lambda i: (0, i, 0))


def _layer_norm(x, g, b):
    mu = jnp.mean(x, axis=-1, keepdims=True)
    var = jnp.mean((x - mu) ** 2, axis=-1, keepdims=True)
    return (x - mu) * lax.rsqrt(var + 1e-5) * g + b


def _dinv_of(deg_ref):
    deg = deg_ref[0, :, 0] + deg_ref[1, :, 0] + 2.0
    return lax.rsqrt(deg)[:, None]


def _emb_body(x_ref, w_ref, b_ref, o_ref):
    o_ref[...] = (
        jnp.dot(x_ref[...], w_ref[...], preferred_element_type=jnp.float32)
        + b_ref[...]
    )


def _pre_body(h_ref, deg_ref, ng_ref, nb_ref, win_ref, bin_ref,
              sg_ref, sb_ref, wgcn_ref, u_ref, zs_ref):
    dinv = _dinv_of(deg_ref)
    t = _layer_norm(h_ref[...], ng_ref[...], nb_ref[...])
    a = (
        jnp.dot(t, win_ref[...], preferred_element_type=jnp.float32)
        + bin_ref[...]
    )
    u = 0.5 * a * (1.0 + lax.erf(a * 0.7071067811865476))
    g = _layer_norm(u, sg_ref[...], sb_ref[...])
    z = jnp.dot(g, wgcn_ref[...], preferred_element_type=jnp.float32)
    u_ref[...] = u
    zs_ref[...] = z * dinv


def _post_body(h_ref, u_ref, zs_ref, acc_ref, deg_ref, wout_ref, bout_ref,
               bgcn_ref, o_ref):
    dinv = _dinv_of(deg_ref)
    zs = zs_ref[...]
    gcn = dinv * (acc_ref[0] + acc_ref[1] + 2.0 * zs) + bgcn_ref[...]
    gated = jnp.tanh(gcn) * u_ref[...]
    o_ref[...] = (
        h_ref[...]
        + jnp.dot(gated, wout_ref[...], preferred_element_type=jnp.float32)
        + bout_ref[...]
    )


def _final_body(h_ref, w_ref, b_ref, o_ref):
    o_ref[...] = (
        jnp.dot(h_ref[...], w_ref[...], preferred_element_type=jnp.float32)
        + b_ref[...]
    )


_emb = pl.pallas_call(
    _emb_body,
    grid=(GRID,),
    in_specs=[_rows_spec(), _full_spec((HID, HID)), _full_spec((1, HID))],
    out_specs=_rows_spec(),
    out_shape=jax.ShapeDtypeStruct((N, HID), jnp.float32),
)

_pre = pl.pallas_call(
    _pre_body,
    grid=(GRID,),
    in_specs=[
        _rows_spec(), _deg_spec(),
        _full_spec((1, HID)), _full_spec((1, HID)),
        _full_spec((HID, HID)), _full_spec((1, HID)),
        _full_spec((1, HID)), _full_spec((1, HID)),
        _full_spec((HID, HID)),
    ],
    out_specs=[_rows_spec(), _rows_spec()],
    out_shape=[
        jax.ShapeDtypeStruct((N, HID), jnp.float32),
        jax.ShapeDtypeStruct((N, HID), jnp.float32),
    ],
)

_post = pl.pallas_call(
    _post_body,
    grid=(GRID,),
    in_specs=[
        _rows_spec(), _rows_spec(), _rows_spec(),
        pl.BlockSpec((NC, RB, HID), lambda i: (0, i, 0)),
        _deg_spec(),
        _full_spec((HID, HID)), _full_spec((1, HID)), _full_spec((1, HID)),
    ],
    out_specs=_rows_spec(),
    out_shape=jax.ShapeDtypeStruct((N, HID), jnp.float32),
)

_final = pl.pallas_call(
    _final_body,
    grid=(GRID,),
    in_specs=[_rows_spec(), _full_spec((HID, C)), _full_spec((1, C))],
    out_specs=_rows_spec(C),
    out_shape=jax.ShapeDtypeStruct((N, C), jnp.float32),
)


def kernel(x, params, edge_index):
    f32 = jnp.float32
    src = edge_index[0].astype(jnp.int32)
    dst = edge_index[1].astype(jnp.int32)
    pad = EP - E
    srcp = jnp.concatenate([src, jnp.zeros((pad,), jnp.int32)]).reshape(
        EP // CHUNK, CHUNK)
    dstp = jnp.concatenate([dst, jnp.full((pad,), N, jnp.int32)]).reshape(
        EP // CHUNK, CHUNK)

    ones_dw = jnp.ones((CHUNK, DW), f32)
    zeros_dw = jnp.zeros((RPT, DW), f32)
    zeros_h = jnp.zeros((RPT, HID), f32)

    degp = _sc_degree(dstp, ones_dw, zeros_dw)[:, :N, :]

    p = params
    h = _emb(x, p['Wemb'].T, p['bemb'][None, :])
    for lp in p['layers']:
        u, zs = _pre(
            h, degp,
            lp['norm_g'][None, :], lp['norm_b'][None, :],
            lp['Win'].T, lp['bin'][None, :],
            lp['sgu_norm_g'][None, :], lp['sgu_norm_b'][None, :],
            lp['Wgcn'].T,
        )
        acc = _sc_gather_scatter(zs, srcp, dstp, zeros_h)[:, :N, :]
        h = _post(
            h, u, zs, acc, degp,
            lp['Wout'].T, lp['bout'][None, :], lp['bgcn'][None, :],
        )
    return _final(h, p['Wlin'].T, p['blin'][None, :])
